# baseline (speedup 1.0000x reference)
import functools

import jax
import jax.numpy as jnp
from jax import lax
from jax.experimental import pallas as pl
from jax.experimental.pallas import tpu as pltpu
from jax.experimental.pallas import tpu_sc as plsc

_D = 64
_NPOS = 200
_LANES = 16
_NW = 32
_B_SC = 1024
_BB = 8


def _pe_sc_kernel(enc_hbm, idx_hbm, tab_hbm, out_hbm,
                  tab_v, idx_v, in_v, out_v,
                  sem_tab, sem_in0, sem_in1, sem_out0, sem_out1):
    batch = enc_hbm.shape[0]
    hist = enc_hbm.shape[1]
    n_chunks = _B_SC // _NW

    wid = lax.axis_index("s") * 2 + lax.axis_index("c")
    b0 = (batch - _B_SC) + wid * n_chunks

    sem_in = (sem_in0, sem_in1)
    sem_out = (sem_out0, sem_out1)

    pltpu.make_async_copy(tab_hbm, tab_v, sem_tab).start()

    def in_copies(g, s):
        b = b0 + g
        return [
            pltpu.make_async_copy(
                enc_hbm.at[b, :, :], in_v.at[s], sem_in[s]),
            pltpu.make_async_copy(
                idx_hbm.at[b, :], idx_v.at[s], sem_in[s]),
        ]

    def out_copy(g, s):
        b = b0 + g
        return pltpu.make_async_copy(
            out_v.at[s], out_hbm.at[0, b, :, :], sem_out[s])

    def start_in(g, s):
        for cp in in_copies(g, s):
            cp.start()

    def wait_in(g, s):
        for cp in in_copies(g, s):
            cp.wait()

    start_in(0, 0)
    start_in(1, 1)
    pltpu.make_async_copy(tab_hbm, tab_v, sem_tab).wait()

    n_full = hist // _LANES
    rem = hist - n_full * _LANES

    def add_rows(s, r0, iv, k_lo, k_hi):
        for k in range(k_lo, k_hi, 2):
            ia = iv[k]
            ib = iv[k + 1]
            nj = _D // _LANES
            ta = [tab_v[ia, pl.ds(j * _LANES, _LANES)] for j in range(nj)]
            tb = [tab_v[ib, pl.ds(j * _LANES, _LANES)] for j in range(nj)]
            ea = [in_v[s, r0 + k, pl.ds(j * _LANES, _LANES)]
                  for j in range(nj)]
            eb = [in_v[s, r0 + k + 1, pl.ds(j * _LANES, _LANES)]
                  for j in range(nj)]
            for j in range(nj):
                out_v[s, r0 + k, pl.ds(j * _LANES, _LANES)] = ea[j] + ta[j]
            for j in range(nj):
                out_v[s, r0 + k + 1, pl.ds(j * _LANES, _LANES)] = (
                    eb[j] + tb[j])

    def do_chunk(g, s):
        wait_in(g, s)

        @pl.when(g >= 2)
        def _():
            out_copy(g - 2, s).wait()

        def group_body(gr, carry):
            r0 = gr * _LANES
            iv = idx_v[s, pl.ds(r0, _LANES)]
            add_rows(s, r0, iv, 0, _LANES)
            return carry

        lax.fori_loop(0, n_full, group_body, 0)

        iv = idx_v[s, pl.ds(hist - _LANES, _LANES)]
        add_rows(s, hist - _LANES, iv, _LANES - rem, _LANES)

        out_copy(g, s).start()

        @pl.when(g + 2 < n_chunks)
        def _():
            start_in(g + 2, s)

    def pair_body(g2, carry):
        do_chunk(2 * g2, 0)
        do_chunk(2 * g2 + 1, 1)
        return carry

    lax.fori_loop(0, n_chunks // 2, pair_body, 0)

    out_copy(n_chunks - 2, 0).wait()
    out_copy(n_chunks - 1, 1).wait()


def _pe_tc_kernel(enc_ref, idx_ref, tab_ref, init_ref, out_ref):
    del init_ref
    idx = idx_ref[...]
    iota = lax.broadcasted_iota(jnp.int32, (1, 1, _NPOS), 2)
    onehot = (idx[:, :, None] == iota).astype(jnp.float32)
    gathered = lax.dot_general(
        onehot, tab_ref[...], (((2,), (0,)), ((), ())),
        preferred_element_type=jnp.float32)
    res = enc_ref[...] + gathered
    out_ref[...] = res.reshape(out_ref.shape)


def kernel(enc_input, ranking, pos_table):
    b, l, d = enc_input.shape
    idx = ranking.astype(jnp.int32)
    tab = pos_table.reshape(_NPOS, d)

    mesh = plsc.VectorSubcoreMesh(core_axis_name="c", subcore_axis_name="s")
    sc_run = pl.kernel(
        _pe_sc_kernel,
        compiler_params=pltpu.CompilerParams(use_tc_tiling_on_sc=True),
        out_type=jax.ShapeDtypeStruct((1, b, l, d), jnp.float32),
        mesh=mesh,
        scratch_types=[
            pltpu.VMEM((_NPOS, d), jnp.float32),
            pltpu.VMEM((2, l), jnp.int32),
            pltpu.VMEM((2, l, d), jnp.float32),
            pltpu.VMEM((2, l, d), jnp.float32),
            pltpu.SemaphoreType.DMA,
            pltpu.SemaphoreType.DMA,
            pltpu.SemaphoreType.DMA,
            pltpu.SemaphoreType.DMA,
            pltpu.SemaphoreType.DMA,
        ],
    )
    sc_out = sc_run(enc_input, idx, tab)

    b_tc = b - _B_SC
    tc_run = pl.pallas_call(
        _pe_tc_kernel,
        grid=(b_tc // _BB,),
        in_specs=[
            pl.BlockSpec((_BB, l, d), lambda i: (i, 0, 0)),
            pl.BlockSpec((_BB, l), lambda i: (i, 0)),
            pl.BlockSpec((_NPOS, d), lambda i: (0, 0)),
            pl.BlockSpec(memory_space=pltpu.MemorySpace.HBM),
        ],
        out_specs=pl.BlockSpec((1, _BB, l, d), lambda i: (0, i, 0, 0)),
        out_shape=jax.ShapeDtypeStruct((1, b, l, d), jnp.float32),
        input_output_aliases={3: 0},
    )
    return tc_run(enc_input, idx, tab, sc_out)


# speedup vs baseline: 1.1920x; 1.1920x over previous
import functools

import jax
import jax.numpy as jnp
from jax import lax
from jax.experimental import pallas as pl
from jax.experimental.pallas import tpu as pltpu
from jax.experimental.pallas import tpu_sc as plsc

_D = 64
_NPOS = 200
_LANES = 16
_NW = 32


def _pe_kernel(enc_hbm, idx_hbm, tab_hbm, out_hbm,
               tab_v, idx_v, in_v, out_v,
               sem_tab, sem_in0, sem_in1, sem_out0, sem_out1):
    batch = enc_hbm.shape[0]
    hist = enc_hbm.shape[1]
    n_chunks = batch // _NW

    wid = lax.axis_index("s") * 2 + lax.axis_index("c")
    b0 = wid * n_chunks

    sem_in = (sem_in0, sem_in1)
    sem_out = (sem_out0, sem_out1)

    pltpu.make_async_copy(tab_hbm, tab_v, sem_tab).start()

    def in_copies(g, s):
        b = b0 + g
        return [
            pltpu.make_async_copy(
                enc_hbm.at[b, :, :], in_v.at[s], sem_in[s]),
            pltpu.make_async_copy(
                idx_hbm.at[b, :], idx_v.at[s], sem_in[s]),
        ]

    def out_copy(g, s):
        b = b0 + g
        return pltpu.make_async_copy(
            out_v.at[s], out_hbm.at[0, b, :, :], sem_out[s])

    def start_in(g, s):
        for cp in in_copies(g, s):
            cp.start()

    def wait_in(g, s):
        for cp in in_copies(g, s):
            cp.wait()

    start_in(0, 0)
    start_in(1, 1)
    pltpu.make_async_copy(tab_hbm, tab_v, sem_tab).wait()

    n_full = hist // _LANES
    rem = hist - n_full * _LANES

    def add_rows(s, r0, iv, k_lo, k_hi):
        for k in range(k_lo, k_hi, 2):
            ia = iv[k]
            ib = iv[k + 1]
            nj = _D // _LANES
            ta = [tab_v[ia, pl.ds(j * _LANES, _LANES)] for j in range(nj)]
            tb = [tab_v[ib, pl.ds(j * _LANES, _LANES)] for j in range(nj)]
            ea = [in_v[s, r0 + k, pl.ds(j * _LANES, _LANES)]
                  for j in range(nj)]
            eb = [in_v[s, r0 + k + 1, pl.ds(j * _LANES, _LANES)]
                  for j in range(nj)]
            for j in range(nj):
                out_v[s, r0 + k, pl.ds(j * _LANES, _LANES)] = ea[j] + ta[j]
            for j in range(nj):
                out_v[s, r0 + k + 1, pl.ds(j * _LANES, _LANES)] = (
                    eb[j] + tb[j])

    def do_chunk(g, s):
        wait_in(g, s)

        @pl.when(g >= 2)
        def _():
            out_copy(g - 2, s).wait()

        def group_body(gr, carry):
            r0 = gr * _LANES
            iv = idx_v[s, pl.ds(r0, _LANES)]
            add_rows(s, r0, iv, 0, _LANES)
            return carry

        lax.fori_loop(0, n_full, group_body, 0)

        iv = idx_v[s, pl.ds(hist - _LANES, _LANES)]
        add_rows(s, hist - _LANES, iv, _LANES - rem, _LANES)

        out_copy(g, s).start()

        @pl.when(g + 2 < n_chunks)
        def _():
            start_in(g + 2, s)

    def pair_body(g2, carry):
        do_chunk(2 * g2, 0)
        do_chunk(2 * g2 + 1, 1)
        return carry

    lax.fori_loop(0, n_chunks // 2, pair_body, 0)

    out_copy(n_chunks - 2, 0).wait()
    out_copy(n_chunks - 1, 1).wait()


def kernel(enc_input, ranking, pos_table):
    b, l, d = enc_input.shape
    idx = ranking.astype(jnp.int32)
    tab = pos_table.reshape(_NPOS, d)

    mesh = plsc.VectorSubcoreMesh(core_axis_name="c", subcore_axis_name="s")
    run = pl.kernel(
        _pe_kernel,
        compiler_params=pltpu.CompilerParams(use_tc_tiling_on_sc=True),
        out_type=jax.ShapeDtypeStruct((1, b, l, d), jnp.float32),
        mesh=mesh,
        scratch_types=[
            pltpu.VMEM((_NPOS, d), jnp.float32),
            pltpu.VMEM((2, l), jnp.int32),
            pltpu.VMEM((2, l, d), jnp.float32),
            pltpu.VMEM((2, l, d), jnp.float32),
            pltpu.SemaphoreType.DMA,
            pltpu.SemaphoreType.DMA,
            pltpu.SemaphoreType.DMA,
            pltpu.SemaphoreType.DMA,
            pltpu.SemaphoreType.DMA,
        ],
    )
    return run(enc_input, idx, tab)
